# baseline (speedup 1.0000x reference)
import functools

import jax
import jax.numpy as jnp
from jax import lax
from jax.experimental import pallas as pl
from jax.experimental.pallas import tpu as pltpu
from jax.experimental.pallas import tpu_sc as plsc

N = 10000
IN = 128
FH = 32
E = 160000

NUM_TILES = 16
CHUNK = 128
ROWS_PER_TILE = 80
GROUP = 8
EP = NUM_TILES * ROWS_PER_TILE * CHUNK
EP_ROWS = EP // CHUNK
NP = 10112
OUT_ROWS_PER_TILE = NP // NUM_TILES
W = 128

_MESH = plsc.VectorSubcoreMesh(core_axis_name="c", subcore_axis_name="s")


@functools.partial(
    pl.kernel,
    out_type=[
        jax.ShapeDtypeStruct((NP, W), jnp.float32),
        jax.ShapeDtypeStruct((NP, W), jnp.float32),
    ],
    mesh=_MESH,
    scratch_types=[
        pltpu.VMEM((2, GROUP, CHUNK), jnp.int32),
        pltpu.VMEM((2, GROUP, CHUNK), jnp.int32),
        pltpu.VMEM((CHUNK, W), jnp.float32),
        pltpu.VMEM((CHUNK, W), jnp.float32),
        pltpu.VMEM_SHARED((NP, W), jnp.float32),
        pltpu.SemaphoreType.DMA,
        pltpu.SemaphoreType.DMA,
        pltpu.SemaphoreType.DMA,
        pltpu.SemaphoreType.DMA,
    ],
)
def _agg(feat, srcp, dstp, srcn, dstn, zeros_hbm, outp, outn,
         srcb, dstb, rows_a, rows_b, accum, sem_ga, sem_gb, sem_sa, sem_sb):
    c = lax.axis_index("c")
    s = lax.axis_index("s")
    r0 = s * OUT_ROWS_PER_TILE
    pltpu.sync_copy(zeros_hbm.at[pl.ds(r0, OUT_ROWS_PER_TILE)],
                    accum.at[pl.ds(r0, OUT_ROWS_PER_TILE)])
    plsc.subcore_barrier()

    def run(src, dst, out):
        base = s * ROWS_PER_TILE

        def load_group(g):
            pltpu.sync_copy(src.at[pl.ds(base + g * GROUP, GROUP)],
                            srcb.at[g % 2])
            pltpu.sync_copy(dst.at[pl.ds(base + g * GROUP, GROUP)],
                            dstb.at[g % 2])

        def issue_gather(j):
            p = (j // GROUP) % 2
            jj = j % GROUP
            half = CHUNK // 2

            @pl.when(j % 2 == 0)
            def _():
                pltpu.async_copy(feat.at[srcb.at[p, jj, pl.ds(0, half)]],
                                 rows_a.at[pl.ds(0, half)], sem_ga)
                pltpu.async_copy(feat.at[srcb.at[p, jj, pl.ds(half, half)]],
                                 rows_a.at[pl.ds(half, half)], sem_ga)

            @pl.when(j % 2 == 1)
            def _():
                pltpu.async_copy(feat.at[srcb.at[p, jj, pl.ds(0, half)]],
                                 rows_b.at[pl.ds(0, half)], sem_gb)
                pltpu.async_copy(feat.at[srcb.at[p, jj, pl.ds(half, half)]],
                                 rows_b.at[pl.ds(half, half)], sem_gb)

        load_group(0)
        issue_gather(0)

        def drain_scatter(parity_sem, parity_rows):
            pltpu.make_async_copy(parity_rows, accum.at[dstb.at[0, 0]],
                                  parity_sem).wait()

        def body(j, tok):
            jn = j + 1

            @pl.when(jnp.logical_and(jn % GROUP == 0, jn < ROWS_PER_TILE))
            def _():
                load_group(jn // GROUP)

            @pl.when(jnp.logical_and(jn < ROWS_PER_TILE, j >= 1))
            def _():
                @pl.when(jn % 2 == 0)
                def _():
                    drain_scatter(sem_sa, rows_a)

                @pl.when(jn % 2 == 1)
                def _():
                    drain_scatter(sem_sb, rows_b)

            @pl.when(jn < ROWS_PER_TILE)
            def _():
                issue_gather(jn)

            p = (j // GROUP) % 2

            @pl.when(j % 2 == 0)
            def _():
                pltpu.make_async_copy(feat.at[srcb.at[0, 0]], rows_a,
                                      sem_ga).wait()
                pltpu.async_copy(rows_a, accum.at[dstb.at[p, j % GROUP]],
                                 sem_sa, add=True)

            @pl.when(j % 2 == 1)
            def _():
                pltpu.make_async_copy(feat.at[srcb.at[0, 0]], rows_b,
                                      sem_gb).wait()
                pltpu.async_copy(rows_b, accum.at[dstb.at[p, j % GROUP]],
                                 sem_sb, add=True)

            return tok

        lax.fori_loop(0, ROWS_PER_TILE, body, 0)
        drain_scatter(sem_sa, rows_a)
        drain_scatter(sem_sb, rows_b)
        plsc.subcore_barrier()
        pltpu.sync_copy(accum.at[pl.ds(r0, OUT_ROWS_PER_TILE)],
                        out.at[pl.ds(r0, OUT_ROWS_PER_TILE)])

    @pl.when(c == 0)
    def _():
        run(srcp, dstp, outp)

    @pl.when(c == 1)
    def _():
        run(srcn, dstn, outn)


def _tc1_body(x_ref, w_ref, b_ref, ws_ref, bs_ref, f_ref, sc_ref):
    xv = x_ref[...]
    f_ref[...] = jnp.dot(xv, w_ref[...],
                         preferred_element_type=jnp.float32) + b_ref[...]
    sc_ref[...] = jnp.dot(xv, ws_ref[...],
                          preferred_element_type=jnp.float32) + bs_ref[...]


def _tc2_body(sp_ref, sn_ref, self_ref, zcat_ref, rinv_ref):
    sp = sp_ref[...]
    sn = sn_ref[...]
    rp = 1.0 / jnp.maximum(sp[:, FH:FH + 1], 1.0)
    rn = 1.0 / jnp.maximum(sn[:, 40 + FH:40 + FH + 1], 1.0)
    zp = sp[:, :FH] * rp + self_ref[:, :FH]
    zn = sn[:, 40:40 + FH] * rn + self_ref[:, FH:]
    zcat_ref[...] = jnp.concatenate(
        [zp, zn, jnp.zeros((zp.shape[0], W - 2 * FH), jnp.float32)], axis=1)
    rinv_ref[...] = jnp.concatenate(
        [rp, rn, jnp.zeros((rp.shape[0], 6), jnp.float32)], axis=1)


def _tc3_body(s2p_ref, s2n_ref, zcat_ref, rinv_ref, wa_ref, wb_ref, wc_ref,
              bb_ref, z_ref):
    rp = rinv_ref[:, 0:1]
    rn = rinv_ref[:, 1:2]
    g1 = s2p_ref[...] * rp
    g2 = s2n_ref[...] * rn
    z_ref[...] = (
        jnp.dot(g1, wa_ref[...], preferred_element_type=jnp.float32)
        + jnp.dot(g2, wb_ref[...], preferred_element_type=jnp.float32)
        + jnp.dot(zcat_ref[...], wc_ref[...],
                  preferred_element_type=jnp.float32)
        + bb_ref[...])


_RB = 2000


def _row_spec(width):
    return pl.BlockSpec((_RB, width), lambda i: (i, 0))


def _full_spec(shape):
    return pl.BlockSpec(shape, lambda i: (0,) * len(shape))


def _pad_edges(idx, fill):
    pad = EP - E
    return jnp.concatenate(
        [idx, jnp.full((pad,), fill, jnp.int32)]).reshape(EP_ROWS, CHUNK)


def kernel(x, pos_edge_index, neg_edge_index,
           W1p, b1p, W1n, b1n, W2p, b2p, W2n, b2n):
    f32 = jnp.float32
    A1p, B1p = W1p[:IN], W1p[IN:]
    A1n, B1n = W1n[:IN], W1n[IN:]
    W1cat = jnp.zeros((IN, W), f32).at[:, 0:FH].set(A1p).at[:, 40:40 + FH].set(A1n)
    b1cat = jnp.zeros((1, W), f32).at[0, FH].set(1.0).at[0, 40 + FH].set(1.0)
    Ws = jnp.concatenate([B1p, B1n], axis=1)
    bs = jnp.concatenate([b1p, b1n])[None, :]
    Wa = (jnp.zeros((W, 2 * FH), f32)
          .at[0:FH, 0:FH].set(W2p[0:FH])
          .at[FH:2 * FH, FH:].set(W2n[0:FH]))
    Wb = (jnp.zeros((W, 2 * FH), f32)
          .at[0:FH, FH:].set(W2n[FH:2 * FH])
          .at[FH:2 * FH, 0:FH].set(W2p[FH:2 * FH]))
    Wc = (jnp.zeros((W, 2 * FH), f32)
          .at[0:FH, 0:FH].set(W2p[2 * FH:])
          .at[FH:2 * FH, FH:].set(W2n[2 * FH:]))
    bbig = jnp.concatenate([b2p, b2n])[None, :]

    srcp = _pad_edges(pos_edge_index[0], 0)
    dstp = _pad_edges(pos_edge_index[1], N)
    srcn = _pad_edges(neg_edge_index[0], 0)
    dstn = _pad_edges(neg_edge_index[1], N)
    zeros_w = jnp.zeros((NP, W), f32)

    feat1, selfcat = pl.pallas_call(
        _tc1_body,
        grid=(N // _RB,),
        in_specs=[
            _row_spec(IN),
            _full_spec((IN, W)), _full_spec((1, W)),
            _full_spec((IN, 2 * FH)), _full_spec((1, 2 * FH)),
        ],
        out_specs=[_row_spec(W), _row_spec(2 * FH)],
        out_shape=[
            jax.ShapeDtypeStruct((N, W), f32),
            jax.ShapeDtypeStruct((N, 2 * FH), f32),
        ],
    )(x, W1cat, b1cat, Ws, bs)

    sum1p, sum1n = _agg(feat1, srcp, dstp, srcn, dstn, zeros_w)

    zcat, rinv = pl.pallas_call(
        _tc2_body,
        grid=(N // _RB,),
        in_specs=[_row_spec(W), _row_spec(W), _row_spec(2 * FH)],
        out_specs=[_row_spec(W), _row_spec(8)],
        out_shape=[
            jax.ShapeDtypeStruct((N, W), f32),
            jax.ShapeDtypeStruct((N, 8), f32),
        ],
    )(sum1p, sum1n, selfcat)

    sum2p, sum2n = _agg(zcat, srcp, dstp, srcn, dstn, zeros_w)

    z = pl.pallas_call(
        _tc3_body,
        grid=(N // _RB,),
        in_specs=[
            _row_spec(W), _row_spec(W), _row_spec(W), _row_spec(8),
            _full_spec((W, 2 * FH)), _full_spec((W, 2 * FH)),
            _full_spec((W, 2 * FH)), _full_spec((1, 2 * FH)),
        ],
        out_specs=[_row_spec(2 * FH)],
        out_shape=[jax.ShapeDtypeStruct((N, 2 * FH), f32)],
    )(sum2p, sum2n, zcat, rinv, Wa, Wb, Wc, bbig)[0]
    return z


# speedup vs baseline: 1.0105x; 1.0105x over previous
import functools

import jax
import jax.numpy as jnp
from jax import lax
from jax.experimental import pallas as pl
from jax.experimental.pallas import tpu as pltpu
from jax.experimental.pallas import tpu_sc as plsc

N = 10000
IN = 128
FH = 32
E = 160000

NUM_TILES = 16
CHUNK = 128
ROWS_PER_TILE = 80
GROUP = 8
EP = NUM_TILES * ROWS_PER_TILE * CHUNK
EP_ROWS = EP // CHUNK
NP = 10112
OUT_ROWS_PER_TILE = NP // NUM_TILES
W = 128

_MESH = plsc.VectorSubcoreMesh(core_axis_name="c", subcore_axis_name="s")


@functools.partial(
    pl.kernel,
    out_type=[
        jax.ShapeDtypeStruct((NP, W), jnp.float32),
        jax.ShapeDtypeStruct((NP, W), jnp.float32),
    ],
    mesh=_MESH,
    scratch_types=[
        pltpu.VMEM((3, GROUP, CHUNK), jnp.int32),
        pltpu.VMEM((3, GROUP, CHUNK), jnp.int32),
        pltpu.VMEM((CHUNK, W), jnp.float32),
        pltpu.VMEM((CHUNK, W), jnp.float32),
        pltpu.VMEM_SHARED((NP, W), jnp.float32),
        pltpu.SemaphoreType.DMA,
        pltpu.SemaphoreType.DMA,
        pltpu.SemaphoreType.DMA,
        pltpu.SemaphoreType.DMA,
        pltpu.SemaphoreType.DMA,
    ],
)
def _agg(feat, srcp, dstp, srcn, dstn, zeros_hbm, outp, outn,
         srcb, dstb, rows_a, rows_b, accum,
         sem_ga, sem_gb, sem_sa, sem_sb, sem_idx):
    c = lax.axis_index("c")
    s = lax.axis_index("s")
    r0 = s * OUT_ROWS_PER_TILE
    pltpu.sync_copy(zeros_hbm.at[pl.ds(r0, OUT_ROWS_PER_TILE)],
                    accum.at[pl.ds(r0, OUT_ROWS_PER_TILE)])
    plsc.subcore_barrier()

    def run(src, dst, out):
        base = s * ROWS_PER_TILE

        def prefetch_group(g):
            pltpu.async_copy(src.at[pl.ds(base + g * GROUP, GROUP)],
                             srcb.at[g % 3], sem_idx)
            pltpu.async_copy(dst.at[pl.ds(base + g * GROUP, GROUP)],
                             dstb.at[g % 3], sem_idx)

        def wait_group(g):
            pltpu.make_async_copy(src.at[pl.ds(base, GROUP)],
                                  srcb.at[g % 3], sem_idx).wait()
            pltpu.make_async_copy(dst.at[pl.ds(base, GROUP)],
                                  dstb.at[g % 3], sem_idx).wait()

        def issue_gather(j):
            p = (j // GROUP) % 3
            jj = j % GROUP
            half = CHUNK // 2

            @pl.when(j % 2 == 0)
            def _():
                pltpu.async_copy(feat.at[srcb.at[p, jj, pl.ds(0, half)]],
                                 rows_a.at[pl.ds(0, half)], sem_ga)
                pltpu.async_copy(feat.at[srcb.at[p, jj, pl.ds(half, half)]],
                                 rows_a.at[pl.ds(half, half)], sem_ga)

            @pl.when(j % 2 == 1)
            def _():
                pltpu.async_copy(feat.at[srcb.at[p, jj, pl.ds(0, half)]],
                                 rows_b.at[pl.ds(0, half)], sem_gb)
                pltpu.async_copy(feat.at[srcb.at[p, jj, pl.ds(half, half)]],
                                 rows_b.at[pl.ds(half, half)], sem_gb)

        prefetch_group(0)
        wait_group(0)
        prefetch_group(1)
        issue_gather(0)

        def drain_scatter(parity_sem, parity_rows):
            pltpu.make_async_copy(parity_rows, accum.at[dstb.at[0, 0]],
                                  parity_sem).wait()

        def body(j, tok):
            jn = j + 1

            @pl.when(jnp.logical_and(jn % GROUP == 0, jn < ROWS_PER_TILE))
            def _():
                g = jn // GROUP
                wait_group(g)

                @pl.when(g + 1 < ROWS_PER_TILE // GROUP)
                def _():
                    prefetch_group(g + 1)

            @pl.when(jnp.logical_and(jn < ROWS_PER_TILE, j >= 1))
            def _():
                @pl.when(jn % 2 == 0)
                def _():
                    drain_scatter(sem_sa, rows_a)

                @pl.when(jn % 2 == 1)
                def _():
                    drain_scatter(sem_sb, rows_b)

            @pl.when(jn < ROWS_PER_TILE)
            def _():
                issue_gather(jn)

            p = (j // GROUP) % 3

            @pl.when(j % 2 == 0)
            def _():
                pltpu.make_async_copy(feat.at[srcb.at[0, 0]], rows_a,
                                      sem_ga).wait()
                pltpu.async_copy(rows_a, accum.at[dstb.at[p, j % GROUP]],
                                 sem_sa, add=True)

            @pl.when(j % 2 == 1)
            def _():
                pltpu.make_async_copy(feat.at[srcb.at[0, 0]], rows_b,
                                      sem_gb).wait()
                pltpu.async_copy(rows_b, accum.at[dstb.at[p, j % GROUP]],
                                 sem_sb, add=True)

            return tok

        lax.fori_loop(0, ROWS_PER_TILE, body, 0)
        drain_scatter(sem_sa, rows_a)
        drain_scatter(sem_sb, rows_b)
        plsc.subcore_barrier()
        pltpu.sync_copy(accum.at[pl.ds(r0, OUT_ROWS_PER_TILE)],
                        out.at[pl.ds(r0, OUT_ROWS_PER_TILE)])

    @pl.when(c == 0)
    def _():
        run(srcp, dstp, outp)

    @pl.when(c == 1)
    def _():
        run(srcn, dstn, outn)


def _tc1_body(x_ref, w_ref, b_ref, ws_ref, bs_ref, f_ref, sc_ref):
    xv = x_ref[...]
    f_ref[...] = jnp.dot(xv, w_ref[...],
                         preferred_element_type=jnp.float32) + b_ref[...]
    sc_ref[...] = jnp.dot(xv, ws_ref[...],
                          preferred_element_type=jnp.float32) + bs_ref[...]


def _tc2_body(sp_ref, sn_ref, self_ref, zcat_ref, rinv_ref):
    sp = sp_ref[...]
    sn = sn_ref[...]
    rp = 1.0 / jnp.maximum(sp[:, FH:FH + 1], 1.0)
    rn = 1.0 / jnp.maximum(sn[:, 40 + FH:40 + FH + 1], 1.0)
    zp = sp[:, :FH] * rp + self_ref[:, :FH]
    zn = sn[:, 40:40 + FH] * rn + self_ref[:, FH:]
    zcat_ref[...] = jnp.concatenate(
        [zp, zn, jnp.zeros((zp.shape[0], W - 2 * FH), jnp.float32)], axis=1)
    rinv_ref[...] = jnp.concatenate(
        [rp, rn, jnp.zeros((rp.shape[0], 6), jnp.float32)], axis=1)


def _tc3_body(s2p_ref, s2n_ref, zcat_ref, rinv_ref, wa_ref, wb_ref, wc_ref,
              bb_ref, z_ref):
    rp = rinv_ref[:, 0:1]
    rn = rinv_ref[:, 1:2]
    g1 = s2p_ref[...] * rp
    g2 = s2n_ref[...] * rn
    z_ref[...] = (
        jnp.dot(g1, wa_ref[...], preferred_element_type=jnp.float32)
        + jnp.dot(g2, wb_ref[...], preferred_element_type=jnp.float32)
        + jnp.dot(zcat_ref[...], wc_ref[...],
                  preferred_element_type=jnp.float32)
        + bb_ref[...])


_RB = 2000


def _row_spec(width):
    return pl.BlockSpec((_RB, width), lambda i: (i, 0))


def _full_spec(shape):
    return pl.BlockSpec(shape, lambda i: (0,) * len(shape))


def _pad_edges(idx, fill):
    pad = EP - E
    return jnp.concatenate(
        [idx, jnp.full((pad,), fill, jnp.int32)]).reshape(EP_ROWS, CHUNK)


def kernel(x, pos_edge_index, neg_edge_index,
           W1p, b1p, W1n, b1n, W2p, b2p, W2n, b2n):
    f32 = jnp.float32
    A1p, B1p = W1p[:IN], W1p[IN:]
    A1n, B1n = W1n[:IN], W1n[IN:]
    W1cat = jnp.zeros((IN, W), f32).at[:, 0:FH].set(A1p).at[:, 40:40 + FH].set(A1n)
    b1cat = jnp.zeros((1, W), f32).at[0, FH].set(1.0).at[0, 40 + FH].set(1.0)
    Ws = jnp.concatenate([B1p, B1n], axis=1)
    bs = jnp.concatenate([b1p, b1n])[None, :]
    Wa = (jnp.zeros((W, 2 * FH), f32)
          .at[0:FH, 0:FH].set(W2p[0:FH])
          .at[FH:2 * FH, FH:].set(W2n[0:FH]))
    Wb = (jnp.zeros((W, 2 * FH), f32)
          .at[0:FH, FH:].set(W2n[FH:2 * FH])
          .at[FH:2 * FH, 0:FH].set(W2p[FH:2 * FH]))
    Wc = (jnp.zeros((W, 2 * FH), f32)
          .at[0:FH, 0:FH].set(W2p[2 * FH:])
          .at[FH:2 * FH, FH:].set(W2n[2 * FH:]))
    bbig = jnp.concatenate([b2p, b2n])[None, :]

    srcp = _pad_edges(pos_edge_index[0], 0)
    dstp = _pad_edges(pos_edge_index[1], N)
    srcn = _pad_edges(neg_edge_index[0], 0)
    dstn = _pad_edges(neg_edge_index[1], N)
    zeros_w = jnp.zeros((NP, W), f32)

    feat1, selfcat = pl.pallas_call(
        _tc1_body,
        grid=(N // _RB,),
        in_specs=[
            _row_spec(IN),
            _full_spec((IN, W)), _full_spec((1, W)),
            _full_spec((IN, 2 * FH)), _full_spec((1, 2 * FH)),
        ],
        out_specs=[_row_spec(W), _row_spec(2 * FH)],
        out_shape=[
            jax.ShapeDtypeStruct((N, W), f32),
            jax.ShapeDtypeStruct((N, 2 * FH), f32),
        ],
    )(x, W1cat, b1cat, Ws, bs)

    sum1p, sum1n = _agg(feat1, srcp, dstp, srcn, dstn, zeros_w)

    zcat, rinv = pl.pallas_call(
        _tc2_body,
        grid=(N // _RB,),
        in_specs=[_row_spec(W), _row_spec(W), _row_spec(2 * FH)],
        out_specs=[_row_spec(W), _row_spec(8)],
        out_shape=[
            jax.ShapeDtypeStruct((N, W), f32),
            jax.ShapeDtypeStruct((N, 8), f32),
        ],
    )(sum1p, sum1n, selfcat)

    sum2p, sum2n = _agg(zcat, srcp, dstp, srcn, dstn, zeros_w)

    z = pl.pallas_call(
        _tc3_body,
        grid=(N // _RB,),
        in_specs=[
            _row_spec(W), _row_spec(W), _row_spec(W), _row_spec(8),
            _full_spec((W, 2 * FH)), _full_spec((W, 2 * FH)),
            _full_spec((W, 2 * FH)), _full_spec((1, 2 * FH)),
        ],
        out_specs=[_row_spec(2 * FH)],
        out_shape=[jax.ShapeDtypeStruct((N, 2 * FH), f32)],
    )(sum2p, sum2n, zcat, rinv, Wa, Wb, Wc, bbig)[0]
    return z
